# Initial kernel scaffold; baseline (speedup 1.0000x reference)
import jax
import jax.numpy as jnp
from jax.experimental import pallas as pl


def kernel(pairs, labels, A, embedding_states, W):
    raise NotImplementedError("write your pallas kernel here")


# speedup vs baseline: 1.6567x; 1.6567x over previous
import functools

import jax
import jax.numpy as jnp
from jax import lax
from jax.experimental import pallas as pl
from jax.experimental.pallas import tpu as pltpu
from jax.experimental.pallas import tpu_sc as plsc

N = 4096
D = 128
NC = 2
NS = 16
NW = NC * NS
LANES = 16
BI = 256
CH = 128


def _l_body(a_ref, x_ref, o_ref):
    a = a_ref[...]
    x = x_ref[...]
    y = jnp.dot(a, x, preferred_element_type=jnp.float32)
    deg = jnp.sum(a, axis=1, keepdims=True)
    o_ref[...] = y / jnp.maximum(deg, 1e-8)


def _l_apply(A, X):
    K = X.shape[1]
    return pl.pallas_call(
        _l_body,
        grid=(N // BI,),
        in_specs=[
            pl.BlockSpec((BI, N), lambda i: (i, 0)),
            pl.BlockSpec((N, K), lambda i: (0, 0)),
        ],
        out_specs=pl.BlockSpec((BI, K), lambda i: (i, 0)),
        out_shape=jax.ShapeDtypeStruct((N, K), jnp.float32),
    )(A, X)


def _comb1_body(y1_ref, y2_ref, w_ref, h_ref, rinv_ref):
    y1 = y1_ref[...]
    y2 = y2_ref[...]
    mix = 0.5 * y1 + 0.25 * y2
    col = lax.broadcasted_iota(jnp.int32, (BI, 2 * D), 1)
    r = jnp.sum(jnp.where(col == D, mix, 0.0), axis=1, keepdims=True)
    rinv = 1.0 / jnp.maximum(r, 1e-8)
    g = mix[:, :D] * rinv
    h_ref[...] = jnp.tanh(jnp.dot(g, w_ref[...],
                                  preferred_element_type=jnp.float32))
    rinv_ref[...] = jnp.broadcast_to(rinv, (BI, D))


def _comb1(Y1, Y2, W0):
    return pl.pallas_call(
        _comb1_body,
        grid=(N // BI,),
        in_specs=[
            pl.BlockSpec((BI, 2 * D), lambda i: (i, 0)),
            pl.BlockSpec((BI, 2 * D), lambda i: (i, 0)),
            pl.BlockSpec((D, D), lambda i: (0, 0)),
        ],
        out_specs=[
            pl.BlockSpec((BI, D), lambda i: (i, 0)),
            pl.BlockSpec((BI, D), lambda i: (i, 0)),
        ],
        out_shape=[
            jax.ShapeDtypeStruct((N, D), jnp.float32),
            jax.ShapeDtypeStruct((N, D), jnp.float32),
        ],
    )(Y1, Y2, W0)


def _comb2_body(y3_ref, y4_ref, rinv_ref, w_ref, h_ref):
    g = (0.5 * y3_ref[...] + 0.25 * y4_ref[...]) * rinv_ref[...]
    h_ref[...] = jnp.tanh(jnp.dot(g, w_ref[...],
                                  preferred_element_type=jnp.float32))


def _comb2(Y3, Y4, rinvb, W1):
    return pl.pallas_call(
        _comb2_body,
        grid=(N // BI,),
        in_specs=[
            pl.BlockSpec((BI, D), lambda i: (i, 0)),
            pl.BlockSpec((BI, D), lambda i: (i, 0)),
            pl.BlockSpec((BI, D), lambda i: (i, 0)),
            pl.BlockSpec((D, D), lambda i: (0, 0)),
        ],
        out_specs=pl.BlockSpec((BI, D), lambda i: (i, 0)),
        out_shape=jax.ShapeDtypeStruct((N, D), jnp.float32),
    )(Y3, Y4, rinvb, W1)


def _pair_loss_sc(table, sidx, didx, labels):
    B = sidx.shape[0]
    per_w = B // NW
    nch = per_w // CH
    mesh = plsc.VectorSubcoreMesh(core_axis_name="c", subcore_axis_name="s",
                                  num_cores=NC, num_subcores=NS)

    @functools.partial(
        pl.kernel,
        out_type=jax.ShapeDtypeStruct((NW, LANES), jnp.float32),
        mesh=mesh,
        scratch_types=[
            pltpu.VMEM((CH,), jnp.int32),
            pltpu.VMEM((CH,), jnp.int32),
            pltpu.VMEM((CH,), jnp.float32),
            pltpu.VMEM((CH, D), jnp.float32),
            pltpu.VMEM((CH, D), jnp.float32),
            pltpu.VMEM((LANES,), jnp.float32),
            pltpu.SemaphoreType.DMA,
            pltpu.SemaphoreType.DMA,
        ],
        compiler_params=pltpu.CompilerParams(needs_layout_passes=False),
    )
    def k(table_hbm, sidx_hbm, didx_hbm, lab_hbm, out_hbm,
          sidx_v, didx_v, lab_v, srows, drows, accv, sem1, sem2):
        wid = lax.axis_index("s") * NC + lax.axis_index("c")
        base = wid * per_w

        def chunk_body(c, loss16):
            off = pl.multiple_of(base + c * CH, CH)
            pltpu.sync_copy(sidx_hbm.at[pl.ds(off, CH)], sidx_v)
            pltpu.sync_copy(didx_hbm.at[pl.ds(off, CH)], didx_v)
            pltpu.sync_copy(lab_hbm.at[pl.ds(off, CH)], lab_v)
            cp1 = pltpu.async_copy(table_hbm.at[sidx_v], srows, sem1)
            cp2 = pltpu.async_copy(table_hbm.at[didx_v], drows, sem2)
            cp1.wait()
            cp2.wait()

            def group_body(g, l16):
                rowi = g * LANES + lax.iota(jnp.int32, LANES)
                acc = jnp.zeros((LANES,), jnp.float32)
                for d in range(D):
                    cols = jnp.full((LANES,), d, jnp.int32)
                    sv = plsc.load_gather(srows, [rowi, cols])
                    dv = plsc.load_gather(drows, [rowi, cols])
                    t = sv - dv
                    acc = acc + t * t
                pred = jnp.exp(acc * (-1.0 / D))
                lab = plsc.load_gather(lab_v, [rowi])
                e = lab - pred
                return l16 + e * e

            return lax.fori_loop(0, CH // LANES, group_body, loss16)

        loss16 = lax.fori_loop(0, nch, chunk_body,
                               jnp.zeros((LANES,), jnp.float32))
        accv[...] = loss16
        pltpu.sync_copy(accv, out_hbm.at[wid])

    return k(table, sidx, didx, labels)


def kernel(pairs, labels, A, embedding_states, W):
    A = A.astype(jnp.float32)
    H = embedding_states.astype(jnp.float32)
    B = pairs.shape[0]

    X0 = jnp.concatenate(
        [H, jnp.ones((N, 1), jnp.float32), jnp.zeros((N, D - 1), jnp.float32)],
        axis=1)
    Y1 = _l_apply(A, X0)
    Y2 = _l_apply(A, Y1)
    H1, rinvb = _comb1(Y1, Y2, W[0])
    Y3 = _l_apply(A, H1)
    Y4 = _l_apply(A, Y3)
    H2 = _comb2(Y3, Y4, rinvb, W[1])

    sidx = pairs[:, 0].astype(jnp.int32)
    didx = pairs[:, 1].astype(jnp.int32)
    partials = _pair_loss_sc(H2, sidx, didx, labels.astype(jnp.float32))
    return jnp.sum(partials) / B


# speedup vs baseline: 1.8933x; 1.1428x over previous
import functools

import jax
import jax.numpy as jnp
from jax import lax
from jax.experimental import pallas as pl
from jax.experimental.pallas import tpu as pltpu
from jax.experimental.pallas import tpu_sc as plsc

N = 4096
D = 128
NC = 2
NS = 16
NW = NC * NS
LANES = 16
BI = 256
CH = 128


def _l_body(a_ref, x_ref, o_ref):
    a = a_ref[...]
    x = x_ref[...]
    y = jnp.dot(a, x, preferred_element_type=jnp.float32)
    deg = jnp.sum(a, axis=1, keepdims=True)
    o_ref[...] = y / jnp.maximum(deg, 1e-8)


def _l_apply(A, X):
    K = X.shape[1]
    return pl.pallas_call(
        _l_body,
        grid=(N // BI,),
        in_specs=[
            pl.BlockSpec((BI, N), lambda i: (i, 0)),
            pl.BlockSpec((N, K), lambda i: (0, 0)),
        ],
        out_specs=pl.BlockSpec((BI, K), lambda i: (i, 0)),
        out_shape=jax.ShapeDtypeStruct((N, K), jnp.float32),
    )(A, X)


def _comb1_body(y1_ref, y2_ref, w_ref, h_ref, rinv_ref):
    y1 = y1_ref[...]
    y2 = y2_ref[...]
    mix = 0.5 * y1 + 0.25 * y2
    col = lax.broadcasted_iota(jnp.int32, (BI, 2 * D), 1)
    r = jnp.sum(jnp.where(col == D, mix, 0.0), axis=1, keepdims=True)
    rinv = 1.0 / jnp.maximum(r, 1e-8)
    g = mix[:, :D] * rinv
    h_ref[...] = jnp.tanh(jnp.dot(g, w_ref[...],
                                  preferred_element_type=jnp.float32))
    rinv_ref[...] = jnp.broadcast_to(rinv, (BI, D))


def _comb1(Y1, Y2, W0):
    return pl.pallas_call(
        _comb1_body,
        grid=(N // BI,),
        in_specs=[
            pl.BlockSpec((BI, 2 * D), lambda i: (i, 0)),
            pl.BlockSpec((BI, 2 * D), lambda i: (i, 0)),
            pl.BlockSpec((D, D), lambda i: (0, 0)),
        ],
        out_specs=[
            pl.BlockSpec((BI, D), lambda i: (i, 0)),
            pl.BlockSpec((BI, D), lambda i: (i, 0)),
        ],
        out_shape=[
            jax.ShapeDtypeStruct((N, D), jnp.float32),
            jax.ShapeDtypeStruct((N, D), jnp.float32),
        ],
    )(Y1, Y2, W0)


def _comb2_body(y3_ref, y4_ref, rinv_ref, w_ref, h_ref):
    g = (0.5 * y3_ref[...] + 0.25 * y4_ref[...]) * rinv_ref[...]
    h_ref[...] = jnp.tanh(jnp.dot(g, w_ref[...],
                                  preferred_element_type=jnp.float32))


def _comb2(Y3, Y4, rinvb, W1):
    return pl.pallas_call(
        _comb2_body,
        grid=(N // BI,),
        in_specs=[
            pl.BlockSpec((BI, D), lambda i: (i, 0)),
            pl.BlockSpec((BI, D), lambda i: (i, 0)),
            pl.BlockSpec((BI, D), lambda i: (i, 0)),
            pl.BlockSpec((D, D), lambda i: (0, 0)),
        ],
        out_specs=pl.BlockSpec((BI, D), lambda i: (i, 0)),
        out_shape=jax.ShapeDtypeStruct((N, D), jnp.float32),
    )(Y3, Y4, rinvb, W1)


def _pair_loss_sc(table, sidx, didx, labels):
    B = sidx.shape[0]
    per_w = B // NW
    nch = per_w // CH
    mesh = plsc.VectorSubcoreMesh(core_axis_name="c", subcore_axis_name="s",
                                  num_cores=NC, num_subcores=NS)

    @functools.partial(
        pl.kernel,
        out_type=jax.ShapeDtypeStruct((NW, LANES), jnp.float32),
        mesh=mesh,
        scratch_types=[
            pltpu.VMEM((per_w,), jnp.int32),
            pltpu.VMEM((per_w,), jnp.int32),
            pltpu.VMEM((per_w,), jnp.float32),
            pltpu.VMEM((CH, D), jnp.float32),
            pltpu.VMEM((CH, D), jnp.float32),
            pltpu.VMEM((CH, D), jnp.float32),
            pltpu.VMEM((CH, D), jnp.float32),
            pltpu.VMEM((LANES,), jnp.float32),
            pltpu.SemaphoreType.DMA,
            pltpu.SemaphoreType.DMA,
        ],
        compiler_params=pltpu.CompilerParams(needs_layout_passes=False),
    )
    def k(table_hbm, sidx_hbm, didx_hbm, lab_hbm, out_hbm,
          sidx_v, didx_v, lab_v, srows0, drows0, srows1, drows1,
          accv, sem0, sem1):
        wid = lax.axis_index("s") * NC + lax.axis_index("c")
        base = pl.multiple_of(wid * per_w, per_w)

        pltpu.sync_copy(sidx_hbm.at[pl.ds(base, per_w)], sidx_v)
        pltpu.sync_copy(didx_hbm.at[pl.ds(base, per_w)], didx_v)
        pltpu.sync_copy(lab_hbm.at[pl.ds(base, per_w)], lab_v)

        def issue(c, srows, drows, sem):
            off = pl.multiple_of(c * CH, CH)
            pltpu.async_copy(table_hbm.at[sidx_v.at[pl.ds(off, CH)]],
                             srows, sem)
            pltpu.async_copy(table_hbm.at[didx_v.at[pl.ds(off, CH)]],
                             drows, sem)

        def drain(srows, drows, sem):
            pltpu.make_async_copy(table_hbm.at[sidx_v.at[pl.ds(0, CH)]],
                                  srows, sem).wait()
            pltpu.make_async_copy(table_hbm.at[didx_v.at[pl.ds(0, CH)]],
                                  drows, sem).wait()

        def compute(c, srows, drows, loss16):
            def group_body(g, l16):
                rowi = g * LANES + lax.iota(jnp.int32, LANES)
                a0 = jnp.zeros((LANES,), jnp.float32)
                a1 = jnp.zeros((LANES,), jnp.float32)
                a2 = jnp.zeros((LANES,), jnp.float32)
                a3 = jnp.zeros((LANES,), jnp.float32)
                accs = [a0, a1, a2, a3]
                for d in range(D):
                    cols = jnp.full((LANES,), d, jnp.int32)
                    sv = plsc.load_gather(srows, [rowi, cols])
                    dv = plsc.load_gather(drows, [rowi, cols])
                    t = sv - dv
                    accs[d % 4] = accs[d % 4] + t * t
                acc = (accs[0] + accs[1]) + (accs[2] + accs[3])
                pred = jnp.exp(acc * (-1.0 / D))
                lab = plsc.load_gather(lab_v, [c * CH + rowi])
                e = lab - pred
                return l16 + e * e

            return lax.fori_loop(0, CH // LANES, group_body, loss16)

        issue(0, srows0, drows0, sem0)

        def pair_body(j, loss16):
            c0 = 2 * j
            c1 = 2 * j + 1
            issue(c1, srows1, drows1, sem1)
            drain(srows0, drows0, sem0)
            loss16 = compute(c0, srows0, drows0, loss16)

            @pl.when(c1 + 1 < nch)
            def _():
                issue(c1 + 1, srows0, drows0, sem0)

            drain(srows1, drows1, sem1)
            return compute(c1, srows1, drows1, loss16)

        loss16 = lax.fori_loop(0, nch // 2, pair_body,
                               jnp.zeros((LANES,), jnp.float32))
        accv[...] = loss16
        pltpu.sync_copy(accv, out_hbm.at[wid])

    return k(table, sidx, didx, labels)


def kernel(pairs, labels, A, embedding_states, W):
    A = A.astype(jnp.float32)
    H = embedding_states.astype(jnp.float32)
    B = pairs.shape[0]

    X0 = jnp.concatenate(
        [H, jnp.ones((N, 1), jnp.float32), jnp.zeros((N, D - 1), jnp.float32)],
        axis=1)
    Y1 = _l_apply(A, X0)
    Y2 = _l_apply(A, Y1)
    H1, rinvb = _comb1(Y1, Y2, W[0])
    Y3 = _l_apply(A, H1)
    Y4 = _l_apply(A, Y3)
    H2 = _comb2(Y3, Y4, rinvb, W[1])

    sidx = pairs[:, 0].astype(jnp.int32)
    didx = pairs[:, 1].astype(jnp.int32)
    partials = _pair_loss_sc(H2, sidx, didx, labels.astype(jnp.float32))
    return jnp.sum(partials) / B


# speedup vs baseline: 4.9253x; 2.6015x over previous
import functools

import jax
import jax.numpy as jnp
from jax import lax
from jax.experimental import pallas as pl
from jax.experimental.pallas import tpu as pltpu
from jax.experimental.pallas import tpu_sc as plsc

N = 4096
D = 128
NC = 2
NS = 16
NW = NC * NS
LANES = 16
BI = 256
CH = 128


def _l_body(a_ref, x_ref, o_ref):
    a = a_ref[...]
    x = x_ref[...]
    y = jnp.dot(a, x, preferred_element_type=jnp.float32)
    deg = jnp.sum(a, axis=1, keepdims=True)
    o_ref[...] = y / jnp.maximum(deg, 1e-8)


def _l_apply(A, X):
    K = X.shape[1]
    return pl.pallas_call(
        _l_body,
        grid=(N // BI,),
        in_specs=[
            pl.BlockSpec((BI, N), lambda i: (i, 0)),
            pl.BlockSpec((N, K), lambda i: (0, 0)),
        ],
        out_specs=pl.BlockSpec((BI, K), lambda i: (i, 0)),
        out_shape=jax.ShapeDtypeStruct((N, K), jnp.float32),
    )(A, X)


def _comb1_body(y1_ref, y2_ref, w_ref, h_ref, rinv_ref):
    y1 = y1_ref[...]
    y2 = y2_ref[...]
    mix = 0.5 * y1 + 0.25 * y2
    col = lax.broadcasted_iota(jnp.int32, (BI, 2 * D), 1)
    r = jnp.sum(jnp.where(col == D, mix, 0.0), axis=1, keepdims=True)
    rinv = 1.0 / jnp.maximum(r, 1e-8)
    g = mix[:, :D] * rinv
    h_ref[...] = jnp.tanh(jnp.dot(g, w_ref[...],
                                  preferred_element_type=jnp.float32))
    rinv_ref[...] = jnp.broadcast_to(rinv, (BI, D))


def _comb1(Y1, Y2, W0):
    return pl.pallas_call(
        _comb1_body,
        grid=(N // BI,),
        in_specs=[
            pl.BlockSpec((BI, 2 * D), lambda i: (i, 0)),
            pl.BlockSpec((BI, 2 * D), lambda i: (i, 0)),
            pl.BlockSpec((D, D), lambda i: (0, 0)),
        ],
        out_specs=[
            pl.BlockSpec((BI, D), lambda i: (i, 0)),
            pl.BlockSpec((BI, D), lambda i: (i, 0)),
        ],
        out_shape=[
            jax.ShapeDtypeStruct((N, D), jnp.float32),
            jax.ShapeDtypeStruct((N, D), jnp.float32),
        ],
    )(Y1, Y2, W0)


def _comb2_body(y3_ref, y4_ref, rinv_ref, w_ref, h_ref):
    g = (0.5 * y3_ref[...] + 0.25 * y4_ref[...]) * rinv_ref[...]
    h_ref[...] = jnp.tanh(jnp.dot(g, w_ref[...],
                                  preferred_element_type=jnp.float32))


def _comb2(Y3, Y4, rinvb, W1):
    return pl.pallas_call(
        _comb2_body,
        grid=(N // BI,),
        in_specs=[
            pl.BlockSpec((BI, D), lambda i: (i, 0)),
            pl.BlockSpec((BI, D), lambda i: (i, 0)),
            pl.BlockSpec((BI, D), lambda i: (i, 0)),
            pl.BlockSpec((D, D), lambda i: (0, 0)),
        ],
        out_specs=pl.BlockSpec((BI, D), lambda i: (i, 0)),
        out_shape=jax.ShapeDtypeStruct((N, D), jnp.float32),
    )(Y3, Y4, rinvb, W1)


def _pair_loss_sc(table, sidx, didx, labels):
    B = sidx.shape[0]
    per_w = B // NW
    nch = per_w // CH
    mesh = plsc.VectorSubcoreMesh(core_axis_name="c", subcore_axis_name="s",
                                  num_cores=NC, num_subcores=NS)

    @functools.partial(
        pl.kernel,
        out_type=jax.ShapeDtypeStruct((NW, LANES), jnp.float32),
        mesh=mesh,
        scratch_types=[
            pltpu.VMEM((per_w,), jnp.int32),
            pltpu.VMEM((per_w,), jnp.int32),
            pltpu.VMEM((per_w,), jnp.float32),
            pltpu.VMEM((CH, D), jnp.float32),
            pltpu.VMEM((CH, D), jnp.float32),
            pltpu.VMEM((CH, D), jnp.float32),
            pltpu.VMEM((CH, D), jnp.float32),
            pltpu.VMEM((LANES,), jnp.float32),
            pltpu.SemaphoreType.DMA,
            pltpu.SemaphoreType.DMA,
        ],
        compiler_params=pltpu.CompilerParams(needs_layout_passes=False),
    )
    def k(table_hbm, sidx_hbm, didx_hbm, lab_hbm, out_hbm,
          sidx_v, didx_v, lab_v, srows0, drows0, srows1, drows1,
          accv, sem0, sem1):
        wid = lax.axis_index("s") * NC + lax.axis_index("c")
        base = pl.multiple_of(wid * per_w, per_w)

        pltpu.sync_copy(sidx_hbm.at[pl.ds(base, per_w)], sidx_v)
        pltpu.sync_copy(didx_hbm.at[pl.ds(base, per_w)], didx_v)
        pltpu.sync_copy(lab_hbm.at[pl.ds(base, per_w)], lab_v)

        def issue(c, srows, drows, sem):
            off = pl.multiple_of(c * CH, CH)
            pltpu.async_copy(table_hbm.at[sidx_v.at[pl.ds(off, CH)]],
                             srows, sem)
            pltpu.async_copy(table_hbm.at[didx_v.at[pl.ds(off, CH)]],
                             drows, sem)

        def drain(srows, drows, sem):
            pltpu.make_async_copy(table_hbm.at[sidx_v.at[pl.ds(0, CH)]],
                                  srows, sem).wait()
            pltpu.make_async_copy(table_hbm.at[didx_v.at[pl.ds(0, CH)]],
                                  drows, sem).wait()

        def compute(c, srows, drows, loss16):
            def group_body(g, l16):
                rowi = g * LANES + lax.iota(jnp.int32, LANES)
                lane = lax.iota(jnp.int32, LANES)
                a0 = jnp.zeros((LANES,), jnp.float32)
                a1 = jnp.zeros((LANES,), jnp.float32)
                a2 = jnp.zeros((LANES,), jnp.float32)
                a3 = jnp.zeros((LANES,), jnp.float32)
                accs = [a0, a1, a2, a3]
                for d in range(D):
                    cols = (lane + d) & (D - 1)
                    sv = plsc.load_gather(srows, [rowi, cols])
                    dv = plsc.load_gather(drows, [rowi, cols])
                    t = sv - dv
                    accs[d % 4] = accs[d % 4] + t * t
                acc = (accs[0] + accs[1]) + (accs[2] + accs[3])
                pred = jnp.exp(acc * (-1.0 / D))
                lab = plsc.load_gather(lab_v, [c * CH + rowi])
                e = lab - pred
                return l16 + e * e

            return lax.fori_loop(0, CH // LANES, group_body, loss16)

        issue(0, srows0, drows0, sem0)

        def pair_body(j, loss16):
            c0 = 2 * j
            c1 = 2 * j + 1
            issue(c1, srows1, drows1, sem1)
            drain(srows0, drows0, sem0)
            loss16 = compute(c0, srows0, drows0, loss16)

            @pl.when(c1 + 1 < nch)
            def _():
                issue(c1 + 1, srows0, drows0, sem0)

            drain(srows1, drows1, sem1)
            return compute(c1, srows1, drows1, loss16)

        loss16 = lax.fori_loop(0, nch // 2, pair_body,
                               jnp.zeros((LANES,), jnp.float32))
        accv[...] = loss16
        pltpu.sync_copy(accv, out_hbm.at[wid])

    return k(table, sidx, didx, labels)


def kernel(pairs, labels, A, embedding_states, W):
    A = A.astype(jnp.float32)
    H = embedding_states.astype(jnp.float32)
    B = pairs.shape[0]

    X0 = jnp.concatenate(
        [H, jnp.ones((N, 1), jnp.float32), jnp.zeros((N, D - 1), jnp.float32)],
        axis=1)
    Y1 = _l_apply(A, X0)
    Y2 = _l_apply(A, Y1)
    H1, rinvb = _comb1(Y1, Y2, W[0])
    Y3 = _l_apply(A, H1)
    Y4 = _l_apply(A, Y3)
    H2 = _comb2(Y3, Y4, rinvb, W[1])

    sidx = pairs[:, 0].astype(jnp.int32)
    didx = pairs[:, 1].astype(jnp.int32)
    partials = _pair_loss_sc(H2, sidx, didx, labels.astype(jnp.float32))
    return jnp.sum(partials) / B
